# baseline (speedup 1.0000x reference)
import functools

import jax
import jax.numpy as jnp
from jax import lax
from jax.experimental import pallas as pl
from jax.experimental.pallas import tpu as pltpu
from jax.experimental.pallas import tpu_sc as plsc

_NUM_CORES = 2
_NUM_SUBCORES = 16
_NW = _NUM_CORES * _NUM_SUBCORES
_NBUF = 2


def _gather_sc(idx_flat, table, chunk, h, d):
    n = idx_flat.shape[0]
    b_per_w = n // _NW
    n_chunks = b_per_w // chunk
    assert n_chunks % _NBUF == 0 and n_chunks >= 2 * _NBUF

    mesh = plsc.VectorSubcoreMesh(
        core_axis_name="c", subcore_axis_name="s",
        num_cores=_NUM_CORES, num_subcores=_NUM_SUBCORES,
    )

    @functools.partial(
        pl.kernel,
        mesh=mesh,
        compiler_params=pltpu.CompilerParams(use_tc_tiling_on_sc=False),
        out_type=jax.ShapeDtypeStruct((n // h, h, d), jnp.float32),
        scratch_types=[
            pltpu.VMEM((_NBUF, chunk), jnp.int32),
            pltpu.VMEM((_NBUF, chunk, d), jnp.float32),
            pltpu.SemaphoreType.DMA((_NBUF,)),
            pltpu.SemaphoreType.DMA((_NBUF,)),
        ],
    )
    def k(idx_hbm, table_hbm, out_hbm, idx_v, rows_v, gsem, osem):
        wid = lax.axis_index("s") * _NUM_CORES + lax.axis_index("c")
        base = wid * b_per_w
        rpc = chunk // h

        def out_copies(cc, b, wait):
            off = base + cc * chunk
            for i in range(rpc):
                cp = pltpu.make_async_copy(
                    rows_v.at[b, pl.ds(i * h, h)],
                    out_hbm.at[off // h + i],
                    osem.at[b],
                )
                if wait:
                    cp.wait()
                else:
                    cp.start()

        def step(cc, b, wait_out):
            off = base + cc * chunk
            if wait_out:
                out_copies(cc, b, wait=True)
            pltpu.sync_copy(idx_hbm.at[pl.ds(off, chunk)], idx_v.at[b])
            pltpu.async_copy(
                table_hbm.at[idx_v.at[b]], rows_v.at[b], gsem.at[b]
            ).wait()
            out_copies(cc, b, wait=False)

        for b in range(_NBUF):
            step(jnp.int32(b), b, wait_out=False)

        def body(r, carry):
            c0 = _NBUF + r * _NBUF
            for b in range(_NBUF):
                step(c0 + b, b, wait_out=True)
            return carry

        lax.fori_loop(0, n_chunks // _NBUF - 1, body, 0)

        for b in range(_NBUF):
            out_copies(jnp.int32(n_chunks - _NBUF + b), b, wait=True)

    return k(idx_flat, table)


@functools.partial(jax.jit, static_argnames=("slabs",))
def _embed(indices, table, slabs):
    nb, h = indices.shape
    d = table.shape[1]
    idx_flat = indices.reshape(nb * h).astype(jnp.int32)
    per = nb * h // slabs
    outs = []
    for s in range(slabs):
        outs.append(
            _gather_sc(
                lax.dynamic_slice_in_dim(idx_flat, s * per, per),
                table, chunk=800, h=h, d=d,
            )
        )
    return jnp.concatenate(outs, axis=0)


def kernel(indices, table):
    return _embed(indices, table, slabs=4)


# speedup vs baseline: 1.0264x; 1.0264x over previous
import functools

import jax
import jax.numpy as jnp
from jax import lax
from jax.experimental import pallas as pl
from jax.experimental.pallas import tpu as pltpu
from jax.experimental.pallas import tpu_sc as plsc

_NUM_CORES = 2
_NUM_SUBCORES = 16
_NW = _NUM_CORES * _NUM_SUBCORES
_NBUF = 2
_HPAD = 208


def _gather_sc(idx_flat, table, chunk, h, d):
    n = idx_flat.shape[0]
    nb = n // h
    b_per_w = n // _NW
    n_chunks = b_per_w // chunk
    assert n_chunks % _NBUF == 0 and n_chunks >= 2 * _NBUF
    assert chunk % h == 0

    mesh = plsc.VectorSubcoreMesh(
        core_axis_name="c", subcore_axis_name="s",
        num_cores=_NUM_CORES, num_subcores=_NUM_SUBCORES,
    )

    @functools.partial(
        pl.kernel,
        mesh=mesh,
        compiler_params=pltpu.CompilerParams(use_tc_tiling_on_sc=False),
        out_type=jax.ShapeDtypeStruct((nb, _HPAD, d), jnp.float32),
        scratch_types=[
            pltpu.VMEM((_NBUF, chunk), jnp.int32),
            pltpu.VMEM((_NBUF, chunk, d), jnp.float32),
            pltpu.SemaphoreType.DMA((_NBUF,)),
            pltpu.SemaphoreType.DMA((_NBUF,)),
        ],
    )
    def k(idx_hbm, table_hbm, out_hbm, idx_v, rows_v, gsem, osem):
        wid = lax.axis_index("s") * _NUM_CORES + lax.axis_index("c")
        base = wid * b_per_w
        rpc = chunk // h

        def out_copies(cc, b, wait):
            off = base + cc * chunk
            for i in range(rpc):
                cp = pltpu.make_async_copy(
                    rows_v.at[b, pl.ds(i * h, h)],
                    out_hbm.at[off // h + i, pl.ds(0, h)],
                    osem.at[b],
                )
                if wait:
                    cp.wait()
                else:
                    cp.start()

        def step(cc, b, wait_out):
            off = base + cc * chunk
            if wait_out:
                out_copies(cc, b, wait=True)
            pltpu.sync_copy(idx_hbm.at[pl.ds(off, chunk)], idx_v.at[b])
            pltpu.async_copy(
                table_hbm.at[idx_v.at[b]], rows_v.at[b], gsem.at[b]
            ).wait()
            out_copies(cc, b, wait=False)

        for b in range(_NBUF):
            step(jnp.int32(b), b, wait_out=False)

        def body(r, carry):
            c0 = _NBUF + r * _NBUF
            for b in range(_NBUF):
                step(c0 + b, b, wait_out=True)
            return carry

        lax.fori_loop(0, n_chunks // _NBUF - 1, body, 0)

        for b in range(_NBUF):
            out_copies(jnp.int32(n_chunks - _NBUF + b), b, wait=True)

    return k(idx_flat, table)


@jax.jit
def _embed(indices, table):
    nb, h = indices.shape
    d = table.shape[1]
    idx_flat = indices.reshape(nb * h).astype(jnp.int32)
    out_pad = _gather_sc(idx_flat, table, chunk=800, h=h, d=d)
    return out_pad[:, :h, :]


def kernel(indices, table):
    return _embed(indices, table)


# speedup vs baseline: 1.4144x; 1.3780x over previous
import functools

import jax
import jax.numpy as jnp
from jax import lax
from jax.experimental import pallas as pl
from jax.experimental.pallas import tpu as pltpu
from jax.experimental.pallas import tpu_sc as plsc

_NUM_CORES = 2
_NUM_SUBCORES = 16
_NW = _NUM_CORES * _NUM_SUBCORES
_NBUF = 2
_BPC = 4


def _gather_sc(indices, table):
    nb, h = indices.shape
    d = table.shape[1]
    b_per_w = nb // _NW
    n_chunks = b_per_w // _BPC
    assert n_chunks % _NBUF == 0 and n_chunks >= 2 * _NBUF

    mesh = plsc.VectorSubcoreMesh(
        core_axis_name="c", subcore_axis_name="s",
        num_cores=_NUM_CORES, num_subcores=_NUM_SUBCORES,
    )

    @functools.partial(
        pl.kernel,
        mesh=mesh,
        compiler_params=pltpu.CompilerParams(use_tc_tiling_on_sc=False),
        out_type=jax.ShapeDtypeStruct((nb, h, d), jnp.float32),
        scratch_types=[
            pltpu.VMEM((_NBUF, _BPC, h), jnp.int32),
            pltpu.VMEM((_NBUF, _BPC, h, d), jnp.float32),
            pltpu.SemaphoreType.DMA((_NBUF,)),
            pltpu.SemaphoreType.DMA((_NBUF,)),
        ],
    )
    def k(idx_hbm, table_hbm, out_hbm, idx_v, rows_v, gsem, osem):
        wid = lax.axis_index("s") * _NUM_CORES + lax.axis_index("c")
        base = wid * b_per_w

        def out_copies(bi, b, wait):
            for j in range(_BPC):
                cp = pltpu.make_async_copy(
                    rows_v.at[b, j], out_hbm.at[bi + j], osem.at[b],
                )
                if wait:
                    cp.wait()
                else:
                    cp.start()

        def step(cc, b, wait_out):
            bi = base + cc * _BPC
            if wait_out:
                out_copies(bi, b, wait=True)
            pltpu.sync_copy(idx_hbm.at[pl.ds(bi, _BPC)], idx_v.at[b])
            for j in range(_BPC):
                pltpu.async_copy(
                    table_hbm.at[idx_v.at[b, j]], rows_v.at[b, j],
                    gsem.at[b],
                )
            for j in range(_BPC):
                pltpu.make_async_copy(
                    table_hbm.at[idx_v.at[b, j]], rows_v.at[b, j],
                    gsem.at[b],
                ).wait()
            out_copies(bi, b, wait=False)

        for b in range(_NBUF):
            step(jnp.int32(b), b, wait_out=False)

        def body(r, carry):
            c0 = _NBUF + r * _NBUF
            for b in range(_NBUF):
                step(c0 + b, b, wait_out=True)
            return carry

        lax.fori_loop(0, n_chunks // _NBUF - 1, body, 0)

        for b in range(_NBUF):
            bi = base + (n_chunks - _NBUF + b) * _BPC
            out_copies(bi, b, wait=True)

    return k(indices, table)


@jax.jit
def _embed(indices, table):
    return _gather_sc(indices.astype(jnp.int32), table)


def kernel(indices, table):
    return _embed(indices, table)


# speedup vs baseline: 2.4808x; 1.7540x over previous
import functools

import jax
import jax.numpy as jnp
from jax import lax
from jax.experimental import pallas as pl
from jax.experimental.pallas import tpu as pltpu
from jax.experimental.pallas import tpu_sc as plsc

_NUM_CORES = 2
_NUM_SUBCORES = 16
_NW = _NUM_CORES * _NUM_SUBCORES
_NBUF = 2
_BPC = 4


def _gather_sc(indices, table):
    nb, h = indices.shape
    d = table.shape[1]
    b_per_w = nb // _NW
    n_chunks = b_per_w // _BPC
    assert n_chunks % _NBUF == 0 and n_chunks >= 2 * _NBUF

    mesh = plsc.VectorSubcoreMesh(
        core_axis_name="c", subcore_axis_name="s",
        num_cores=_NUM_CORES, num_subcores=_NUM_SUBCORES,
    )

    @functools.partial(
        pl.kernel,
        mesh=mesh,
        compiler_params=pltpu.CompilerParams(use_tc_tiling_on_sc=False),
        out_type=jax.ShapeDtypeStruct((nb, h, 2 * d), jnp.float32),
        scratch_types=[
            pltpu.VMEM((_NBUF, _BPC, h), jnp.int32),
            pltpu.VMEM((_NBUF, _BPC, h, d), jnp.float32),
            pltpu.SemaphoreType.DMA((_NBUF,)),
            pltpu.SemaphoreType.DMA((_NBUF,)),
        ],
    )
    def k(idx_hbm, table_hbm, out_hbm, idx_v, rows_v, gsem, osem):
        wid = lax.axis_index("s") * _NUM_CORES + lax.axis_index("c")
        base = wid * b_per_w

        def out_copies(bi, b, wait):
            for j in range(_BPC):
                cp = pltpu.make_async_copy(
                    rows_v.at[b, j],
                    out_hbm.at[bi + j, pl.ds(0, h), pl.ds(0, d)],
                    osem.at[b],
                )
                if wait:
                    cp.wait()
                else:
                    cp.start()

        def step(cc, b, wait_out):
            bi = base + cc * _BPC
            if wait_out:
                out_copies(bi, b, wait=True)
            pltpu.sync_copy(idx_hbm.at[pl.ds(bi, _BPC)], idx_v.at[b])
            for j in range(_BPC):
                pltpu.async_copy(
                    table_hbm.at[idx_v.at[b, j]], rows_v.at[b, j],
                    gsem.at[b],
                )
            for j in range(_BPC):
                pltpu.make_async_copy(
                    table_hbm.at[idx_v.at[b, j]], rows_v.at[b, j],
                    gsem.at[b],
                ).wait()
            out_copies(bi, b, wait=False)

        for b in range(_NBUF):
            step(jnp.int32(b), b, wait_out=False)

        def body(r, carry):
            c0 = _NBUF + r * _NBUF
            for b in range(_NBUF):
                step(c0 + b, b, wait_out=True)
            return carry

        lax.fori_loop(0, n_chunks // _NBUF - 1, body, 0)

        for b in range(_NBUF):
            bi = base + (n_chunks - _NBUF + b) * _BPC
            out_copies(bi, b, wait=True)

    return k(indices, table)


@jax.jit
def _embed(indices, table):
    wide = _gather_sc(indices.astype(jnp.int32), table)
    return wide[:, :, :table.shape[1]]


def kernel(indices, table):
    return _embed(indices, table)
